# baseline (speedup 1.0000x reference)
import functools

import jax
import jax.numpy as jnp
from jax import lax
from jax.experimental import pallas as pl
from jax.experimental.pallas import tpu as pltpu
from jax.experimental.pallas import tpu_sc as plsc

_INFO = plsc.get_sparse_core_info()
_NC, _NS = _INFO.num_cores, _INFO.num_subcores
_NW = _NC * _NS

_NBUF = 4


@functools.partial(jax.jit, static_argnums=(2, 3, 4))
def _sc_gather(table, idx, bpw, seq, dim):
    mesh = plsc.VectorSubcoreMesh(core_axis_name="c", subcore_axis_name="s")

    scratch = (
        [pltpu.VMEM((bpw,), jnp.int32) for _ in range(_NBUF)]
        + [pltpu.VMEM((bpw, dim), jnp.float32) for _ in range(_NBUF)]
        + [pltpu.SemaphoreType.DMA for _ in range(3 * _NBUF)]
    )

    @functools.partial(
        pl.kernel,
        mesh=mesh,
        out_type=jax.ShapeDtypeStruct((seq * 8 * _NW, 8 * bpw, 1), jnp.float32),
        scratch_types=scratch,
        compiler_params=pltpu.CompilerParams(
            use_tc_tiling_on_sc=False, needs_layout_passes=False
        ),
    )
    def k(table_hbm, idx_hbm, out_hbm, *bufs):
        idxt = bufs[:_NBUF]
        rows = bufs[_NBUF : 2 * _NBUF]
        isem = bufs[2 * _NBUF : 3 * _NBUF]
        gsem = bufs[3 * _NBUF : 4 * _NBUF]
        ssem = bufs[4 * _NBUF :]

        wid = lax.axis_index("s") * _NC + lax.axis_index("c")

        def stage(s, b):
            pltpu.async_copy(
                idx_hbm.at[pl.ds(wid * (seq * bpw) + s * bpw, bpw)],
                idxt[b],
                isem[b],
            )

        def fill(s, b):
            pltpu.make_async_copy(
                idx_hbm.at[pl.ds(wid * (seq * bpw) + s * bpw, bpw)],
                idxt[b],
                isem[b],
            ).wait()
            pltpu.async_copy(table_hbm.at[idxt[b]], rows[b], gsem[b])

        def drain(s, b):
            pltpu.make_async_copy(table_hbm.at[idxt[b]], rows[b], gsem[b]).wait()
            for dq in range(8):
                row = (s * 8 + dq) * _NW + wid
                for dr in range(8):
                    pltpu.async_copy(
                        rows[b].at[:, pl.ds(dq * 8 + dr, 1)],
                        out_hbm.at[row, pl.ds(dr * bpw, bpw), :],
                        ssem[b],
                    )

        def store_wait(s, b):
            for d in range(dim):
                pltpu.make_async_copy(
                    rows[b].at[:, pl.ds(0, 1)],
                    out_hbm.at[0, pl.ds(0, bpw), :],
                    ssem[b],
                ).wait()

        stage(0, 0)
        stage(1, 1)
        fill(0, 0)
        for g in range(1, _NBUF):
            if g >= _NBUF - 1:
                store_wait(g - (_NBUF - 1), (g + 1) % _NBUF)
            stage(g + 1, (g + 1) % _NBUF)
            fill(g, g)
            drain(g - 1, g - 1)

        def round_body(r, carry):
            for b in range(_NBUF):
                g = r * _NBUF + b
                store_wait(g - (_NBUF - 1), (b + 1) % _NBUF)

                @pl.when(g + 1 <= seq - 1)
                def _():
                    stage(g + 1, (b + 1) % _NBUF)

                fill(g, b)
                drain(g - 1, (b + _NBUF - 1) % _NBUF)
            return carry

        lax.fori_loop(1, seq // _NBUF, round_body, 0)

        drain(seq - 1, (seq - 1) % _NBUF)
        for i in range(_NBUF - 1):
            g = seq - 1 - i
            store_wait(g, g % _NBUF)

    return k(table, idx)


def kernel(x, table):
    Bt, S = x.shape
    D = table.shape[1]
    assert Bt % _NW == 0 and D == 64
    bpw = Bt // _NW
    idx = (
        x.T.reshape(S, _NW, bpw)
        .transpose(1, 0, 2)
        .reshape(Bt * S)
        .astype(jnp.int32)
    )
    out = _sc_gather(table, idx, bpw, S, D)
    return (
        out.reshape(S, 8, _NW, 8, bpw)
        .transpose(2, 4, 0, 1, 3)
        .reshape(Bt, S, D)
    )


# speedup vs baseline: 78.6189x; 78.6189x over previous
import functools

import jax
import jax.numpy as jnp
from jax import lax
from jax.experimental import pallas as pl
from jax.experimental.pallas import tpu as pltpu
from jax.experimental.pallas import tpu_sc as plsc

_INFO = plsc.get_sparse_core_info()
_NC, _NS = _INFO.num_cores, _INFO.num_subcores
_NW = _NC * _NS

_NBUF = 4
_LAG = 2


@functools.partial(jax.jit, static_argnums=(2, 3))
def _sc_gather(table2, idx, rows_per_w, seq):
    batch = idx.shape[0] // seq
    unit = 2
    n_units = rows_per_w // unit
    mesh = plsc.VectorSubcoreMesh(core_axis_name="c", subcore_axis_name="s")

    scratch = (
        [pltpu.VMEM((unit * seq,), jnp.int32) for _ in range(_NBUF)]
        + [pltpu.VMEM((2 * unit * seq,), jnp.int32) for _ in range(_NBUF)]
        + [pltpu.VMEM((2 * unit * seq, 32), jnp.float32) for _ in range(_NBUF)]
        + [pltpu.SemaphoreType.DMA for _ in range(2 * _NBUF)]
    )

    @functools.partial(
        pl.kernel,
        mesh=mesh,
        out_type=jax.ShapeDtypeStruct((batch, 2 * seq, 32), jnp.float32),
        scratch_types=scratch,
        compiler_params=pltpu.CompilerParams(
            use_tc_tiling_on_sc=False, needs_layout_passes=False
        ),
    )
    def k(table_hbm, idx_hbm, out_hbm, *bufs):
        idxs = bufs[:_NBUF]
        idx2 = bufs[_NBUF : 2 * _NBUF]
        rows = bufs[2 * _NBUF : 3 * _NBUF]
        gsem = bufs[3 * _NBUF : 4 * _NBUF]
        ssem = bufs[4 * _NBUF :]

        wid = lax.axis_index("s") * _NC + lax.axis_index("c")
        row0 = wid * rows_per_w
        lane = lax.iota(jnp.int32, 16)

        def fill(g, b):
            row = row0 + g * unit
            pltpu.sync_copy(idx_hbm.at[pl.ds(row * seq, unit * seq)], idxs[b])

            def expand(v, _):
                q = idxs[b][pl.ds(v * 16, 16)]
                pos = (v * 16 + lane) * 2
                plsc.store_scatter(idx2[b], [pos], q + q)
                plsc.store_scatter(idx2[b], [pos + 1], q + q + 1)
                return _

            lax.fori_loop(0, unit * seq // 16, expand, 0)
            pltpu.async_copy(table_hbm.at[idx2[b]], rows[b], gsem[b])

        def drain(g, b):
            pltpu.make_async_copy(table_hbm.at[idx2[b]], rows[b], gsem[b]).wait()
            row = row0 + g * unit
            pltpu.async_copy(
                rows[b].at[pl.ds(0, 2 * seq)], out_hbm.at[row], ssem[b]
            )
            pltpu.async_copy(
                rows[b].at[pl.ds(2 * seq, 2 * seq)], out_hbm.at[row + 1], ssem[b]
            )

        def store_wait(g, b):
            row = row0 + g * unit
            pltpu.make_async_copy(
                rows[b].at[pl.ds(0, 2 * seq)], out_hbm.at[row], ssem[b]
            ).wait()
            pltpu.make_async_copy(
                rows[b].at[pl.ds(2 * seq, 2 * seq)], out_hbm.at[row + 1], ssem[b]
            ).wait()

        for g in range(_NBUF):
            if g >= _LAG:
                drain(g - _LAG, g - _LAG)
            fill(g, g)

        def round_body(r, carry):
            for b in range(_NBUF):
                g = r * _NBUF + b
                drain(g - _LAG, (b + _NBUF - _LAG) % _NBUF)
                store_wait(g - _NBUF, b)
                fill(g, b)
            return carry

        lax.fori_loop(1, n_units // _NBUF, round_body, 0)

        for i in range(_LAG):
            g = n_units - _LAG + i
            drain(g, g % _NBUF)
        for i in range(_NBUF):
            g = n_units - _NBUF + i
            store_wait(g, g % _NBUF)

    return k(table2, idx)


def kernel(x, table):
    Bt, S = x.shape
    D = table.shape[1]
    idx = x.reshape(Bt * S).astype(jnp.int32)
    table2 = table.reshape(table.shape[0] * 2, D // 2)
    assert Bt % _NW == 0
    out = _sc_gather(table2, idx, Bt // _NW, S)
    return out.reshape(Bt, S, D)


# speedup vs baseline: 99.3430x; 1.2636x over previous
import functools

import jax
import jax.numpy as jnp
from jax import lax
from jax.experimental import pallas as pl
from jax.experimental.pallas import tpu as pltpu
from jax.experimental.pallas import tpu_sc as plsc

_INFO = plsc.get_sparse_core_info()
_NC, _NS = _INFO.num_cores, _INFO.num_subcores
_NW = _NC * _NS

_NBUF = 4
_LAG = 2


@functools.partial(jax.jit, static_argnums=(2, 3, 4))
def _sc_gather(table, idx, b_per_w, chunk, n_chunks):
    D = table.shape[1]
    B = idx.shape[0]
    mesh = plsc.VectorSubcoreMesh(core_axis_name="c", subcore_axis_name="s")

    scratch = (
        [pltpu.VMEM((chunk,), jnp.int32) for _ in range(_NBUF)]
        + [pltpu.VMEM((chunk, D), table.dtype) for _ in range(_NBUF)]
        + [pltpu.SemaphoreType.DMA for _ in range(2 * _NBUF)]
    )

    @functools.partial(
        pl.kernel,
        mesh=mesh,
        out_type=jax.ShapeDtypeStruct((B, D), table.dtype),
        scratch_types=scratch,
        compiler_params=pltpu.CompilerParams(use_tc_tiling_on_sc=False),
    )
    def k(table_hbm, idx_hbm, out_hbm, *bufs):
        idxs = bufs[:_NBUF]
        rows = bufs[_NBUF : 2 * _NBUF]
        gsem = bufs[2 * _NBUF : 3 * _NBUF]
        ssem = bufs[3 * _NBUF :]

        wid = lax.axis_index("s") * _NC + lax.axis_index("c")
        w_base = wid * b_per_w

        def fill(g, b):
            base = w_base + g * chunk
            pltpu.sync_copy(idx_hbm.at[pl.ds(base, chunk)], idxs[b])
            pltpu.async_copy(table_hbm.at[idxs[b]], rows[b], gsem[b])

        def drain(g, b):
            pltpu.make_async_copy(table_hbm.at[idxs[b]], rows[b], gsem[b]).wait()
            base = w_base + g * chunk
            pltpu.async_copy(rows[b], out_hbm.at[pl.ds(base, chunk)], ssem[b])

        def store_wait(g, b):
            base = w_base + g * chunk
            pltpu.make_async_copy(
                rows[b], out_hbm.at[pl.ds(base, chunk)], ssem[b]
            ).wait()

        for g in range(_NBUF):
            if g >= _LAG:
                drain(g - _LAG, g - _LAG)
            fill(g, g)

        def round_body(r, carry):
            for b in range(_NBUF):
                g = r * _NBUF + b
                drain(g - _LAG, (b + _NBUF - _LAG) % _NBUF)
                store_wait(g - _NBUF, b)
                fill(g, b)
            return carry

        lax.fori_loop(1, n_chunks // _NBUF, round_body, 0)

        for i in range(_LAG):
            g = n_chunks - _LAG + i
            drain(g, g % _NBUF)
        for i in range(_NBUF):
            g = n_chunks - _NBUF + i
            store_wait(g, g % _NBUF)

    return k(table, idx)


def kernel(x, table):
    Bt, S = x.shape
    D = table.shape[1]
    B = Bt * S
    idx = x.reshape(B).astype(jnp.int32)
    chunk = 400
    assert B % (_NW * chunk) == 0
    b_per_w = B // _NW
    n_chunks = b_per_w // chunk
    assert n_chunks % _NBUF == 0
    out = _sc_gather(table, idx, b_per_w, chunk, n_chunks)
    return out.reshape(Bt, S, D)
